# baseline (speedup 1.0000x reference)
import jax
import jax.numpy as jnp
from jax.experimental import pallas as pl

_PRE = 2000
_POST = 1000
_THR = 0.7
_VAR = (0.1, 0.1, 0.2, 0.2)


def _decode(anchors, deltas):
    w = anchors[..., 3] - anchors[..., 1]
    h = anchors[..., 2] - anchors[..., 0]
    cx = anchors[..., 1] + 0.5 * w
    cy = anchors[..., 0] + 0.5 * h
    bw = jnp.exp(deltas[..., 3]) * w
    bh = jnp.exp(deltas[..., 2]) * h
    bcx = deltas[..., 1] * w + cx
    bcy = deltas[..., 0] * h + cy
    y1 = bcy - 0.5 * bh
    x1 = bcx - 0.5 * bw
    y2 = bh + y1
    x2 = bw + x1
    return jnp.stack([y1, x1, y2, x2], axis=-1)


def _iou(boxes):
    y1, x1, y2, x2 = boxes[:, 0], boxes[:, 1], boxes[:, 2], boxes[:, 3]
    area = (y2 - y1) * (x2 - x1)
    iy1 = jnp.maximum(y1[:, None], y1[None, :])
    ix1 = jnp.maximum(x1[:, None], x1[None, :])
    iy2 = jnp.minimum(y2[:, None], y2[None, :])
    ix2 = jnp.minimum(x2[:, None], x2[None, :])
    ih = jnp.clip(iy2 - iy1, 0.0)
    iw = jnp.clip(ix2 - ix1, 0.0)
    inter = ih * iw
    union = area[:, None] + area[None, :] - inter
    return inter / jnp.maximum(union, 1e-9)


def _nms_single(boxes, scores):
    n = boxes.shape[0]
    iou = _iou(boxes)

    def body(i, state):
        kept, active = state
        idx = jnp.argmax(active)
        any_active = jnp.any(active)
        kept = kept.at[idx].set(kept[idx] | any_active)
        suppress = iou[idx] > _THR
        new_active = active & (~suppress)
        new_active = new_active.at[idx].set(False)
        active = jnp.where(any_active, new_active, active)
        return kept, active

    kept, _ = jax.lax.fori_loop(0, _POST, body,
                                (jnp.zeros(n, dtype=bool), jnp.ones(n, dtype=bool)))
    masked = jnp.where(kept, scores, -jnp.inf)
    topv, topi = jax.lax.top_k(masked, _POST)
    valid = topv > -jnp.inf
    out_boxes = jnp.where(valid[:, None], boxes[topi], 0.0)
    out_scores = jnp.where(valid, topv, 0.0)
    return out_boxes, out_scores


def _identity_pallas(x):
    def body(x_ref, o_ref):
        o_ref[...] = x_ref[...]
    return pl.pallas_call(
        body, out_shape=jax.ShapeDtypeStruct(x.shape, x.dtype))(x)


def kernel(rpn_bbox_deltas, rpn_labels, anchors):
    B = rpn_bbox_deltas.shape[0]
    N = anchors.shape[0]
    variances = jnp.asarray(_VAR, dtype=rpn_bbox_deltas.dtype)
    deltas = rpn_bbox_deltas.reshape(B, N, 4) * variances
    labels = jax.nn.softmax(_identity_pallas(rpn_labels), axis=-1).reshape(B, N)
    rpn_bboxes = _decode(anchors, deltas)
    top_scores, pre_idx = jax.lax.top_k(labels, _PRE)
    pre_boxes = jnp.take_along_axis(rpn_bboxes, pre_idx[..., None], axis=1)
    out_b, out_s = jax.vmap(lambda b, s: _nms_single(b, s))(pre_boxes, top_scores)
    return out_b, out_s


# speedup vs baseline: 68.2887x; 68.2887x over previous
import functools

import jax
import jax.numpy as jnp
from jax import lax
from jax.experimental import pallas as pl
from jax.experimental.pallas import tpu as pltpu
from jax.experimental.pallas import tpu_sc as plsc

B = 4
N = 20000
NPAD = 20480
CAND = 2048
CSLOT = 4096
PRE = 2000
POST = 1000
THR = 0.7
NW = 32
K2A_ROWS = (B * NPAD) // NW // 128

@functools.cache
def _sc_mesh():
    return plsc.VectorSubcoreMesh(core_axis_name="c", subcore_axis_name="s")


def _k1_body(x_ref, p_ref, pos_ref, aval_ref):
    x = x_ref[...]
    xm = jnp.max(x, axis=1, keepdims=True)
    e = jnp.exp(x - xm)
    s = jnp.sum(e, axis=1, keepdims=True)
    p = e / s
    p_ref[...] = p

    idx = lax.broadcasted_iota(jnp.int32, (B, NPAD), 1)

    def tstep(_, lohi):
        lo, hi = lohi
        mid = (lo + hi) // 2
        t = lax.bitcast_convert_type(mid, jnp.float32)
        cnt = jnp.sum((p > t).astype(jnp.int32), axis=1, keepdims=True)
        ge = cnt >= PRE
        return jnp.where(ge, mid, lo), jnp.where(ge, hi, mid)

    lo0 = jnp.zeros((B, 1), jnp.int32)
    hi0 = jnp.full((B, 1), 0x40000000, jnp.int32)
    _, hi = lax.fori_loop(0, 31, tstep, (lo0, hi0))
    t = lax.bitcast_convert_type(hi, jnp.float32)
    c_gt = jnp.sum((p > t).astype(jnp.int32), axis=1, keepdims=True)
    need = PRE - c_gt
    tie = p == t

    def mstep(_, lohi):
        lo, hi = lohi
        mid = (lo + hi) // 2
        cnt = jnp.sum((tie & (idx <= mid)).astype(jnp.int32), axis=1,
                      keepdims=True)
        ge = cnt >= need
        return jnp.where(ge, lo, mid), jnp.where(ge, mid, hi)

    mlo0 = jnp.full((B, 1), -1, jnp.int32)
    mhi0 = jnp.full((B, 1), NPAD - 1, jnp.int32)
    _, mhi = lax.fori_loop(0, 16, mstep, (mlo0, mhi0))
    sel = (p > t) | (tie & (idx <= mhi))

    r128 = lax.broadcasted_iota(jnp.int32, (128, 128), 0)
    c128 = lax.broadcasted_iota(jnp.int32, (128, 128), 1)
    tri_inc = (r128 <= c128).astype(jnp.float32)
    nblk = NPAD // 128
    rb = lax.broadcasted_iota(jnp.int32, (nblk, nblk), 0)
    cb = lax.broadcasted_iota(jnp.int32, (nblk, nblk), 1)
    tri_exc = (rb < cb).astype(jnp.float32)
    rows = []
    for b in range(B):
        sb = sel[b].astype(jnp.float32).reshape(nblk, 128)
        cinc = jnp.dot(sb, tri_inc, preferred_element_type=jnp.float32)
        bsum = cinc[:, 127]
        bpre = jnp.dot(bsum[None, :], tri_exc,
                       preferred_element_type=jnp.float32)[0]
        rows.append((bpre[:, None] + cinc - 1.0).reshape(NPAD))
    posl = jnp.stack(rows).astype(jnp.int32)
    dump = CAND + (idx % CAND)
    bofs = lax.broadcasted_iota(jnp.int32, (B, NPAD), 0) * CSLOT
    pos_ref[...] = jnp.where(sel, posl, dump) + bofs
    aval_ref[...] = jnp.minimum(idx, N - 1)


_k1 = pl.pallas_call(
    _k1_body,
    out_shape=[
        jax.ShapeDtypeStruct((B, NPAD), jnp.float32),
        jax.ShapeDtypeStruct((B, NPAD), jnp.int32),
        jax.ShapeDtypeStruct((B, NPAD), jnp.int32),
    ],
)


def _k2a_body(pos_hbm, aval_hbm, cand_hbm, posbuf, valbuf, sem):
    w = lax.axis_index("s") * 2 + lax.axis_index("c")
    pltpu.sync_copy(pos_hbm.at[w], posbuf)
    pltpu.sync_copy(aval_hbm.at[w], valbuf)
    descs = [
        pltpu.async_copy(valbuf.at[j], cand_hbm.at[posbuf.at[j]], sem)
        for j in range(K2A_ROWS)
    ]
    for d in descs:
        d.wait()


def _k2b_body(cand3d_hbm, d0_hbm, d1_hbm, d2_hbm, d3_hbm,
              a0_hbm, a1_hbm, a2_hbm, a3_hbm, pflat_hbm,
              cd_flat, ca_flat, cp_flat,
              aidx, gidx, pidx, dbufs, abufs, pbuf, sem):
    w = lax.axis_index("s") * 2 + lax.axis_index("c")
    b = w // 8
    c = w % 8
    pltpu.sync_copy(cand3d_hbm.at[b * 16 + c], aidx)
    for j in range(2):
        for t in range(8):
            v = aidx[j, pl.ds(t * 16, 16)]
            v = jnp.minimum(jnp.maximum(v, 0), N - 1)
            aidx[j, pl.ds(t * 16, 16)] = v
            gidx[j, pl.ds(t * 16, 16)] = v + b * N
            pidx[j, pl.ds(t * 16, 16)] = v + b * NPAD
    dcols = [d0_hbm, d1_hbm, d2_hbm, d3_hbm]
    acols = [a0_hbm, a1_hbm, a2_hbm, a3_hbm]
    descs = []
    for j in range(2):
        for col in range(4):
            descs.append(pltpu.async_copy(
                dcols[col].at[gidx.at[j]],
                dbufs[col].at[pl.ds(j * 128, 128)], sem))
            descs.append(pltpu.async_copy(
                acols[col].at[aidx.at[j]],
                abufs[col].at[pl.ds(j * 128, 128)], sem))
        descs.append(pltpu.async_copy(
            pflat_hbm.at[pidx.at[j]], pbuf.at[pl.ds(j * 128, 128)], sem))
    for d in descs:
        d.wait()
    for col in range(4):
        o = b * 4 * CAND + col * CAND + c * 256
        pltpu.sync_copy(dbufs[col], cd_flat.at[pl.ds(o, 256)])
        pltpu.sync_copy(abufs[col], ca_flat.at[pl.ds(o, 256)])
    pltpu.sync_copy(pbuf, cp_flat.at[pl.ds(b * CAND + c * 256, 256)])


def _k3_body(d_ref, a_ref, p_ref, pos2_ref, bv_ref, pv_ref, m_scr, c_scr):
    d0 = d_ref[0, 0, :] * 0.1
    d1 = d_ref[0, 1, :] * 0.1
    d2 = d_ref[0, 2, :] * 0.2
    d3 = d_ref[0, 3, :] * 0.2
    a0 = a_ref[0, 0, :]
    a1 = a_ref[0, 1, :]
    a2 = a_ref[0, 2, :]
    a3 = a_ref[0, 3, :]
    w = a3 - a1
    h = a2 - a0
    cx = a1 + 0.5 * w
    cy = a0 + 0.5 * h
    bw = jnp.exp(d3) * w
    bh = jnp.exp(d2) * h
    bcx = d1 * w + cx
    bcy = d0 * h + cy
    y1 = bcy - 0.5 * bh
    x1 = bcx - 0.5 * bw
    y2 = bh + y1
    x2 = bw + x1
    area = (y2 - y1) * (x2 - x1)

    pc = p_ref[0, 0, :]
    idx = lax.broadcasted_iota(jnp.int32, (CAND,), 0)
    valid = idx < PRE
    pp = jnp.where(valid, pc, -1.0)

    rb = 256
    for blk in range(CAND // rb):
        sl = slice(blk * rb, (blk + 1) * rb)
        iy1 = jnp.maximum(y1[sl][:, None], y1[None, :])
        ix1 = jnp.maximum(x1[sl][:, None], x1[None, :])
        iy2 = jnp.minimum(y2[sl][:, None], y2[None, :])
        ix2 = jnp.minimum(x2[sl][:, None], x2[None, :])
        ih = jnp.clip(iy2 - iy1, 0.0)
        iw = jnp.clip(ix2 - ix1, 0.0)
        inter = ih * iw
        union = area[sl][:, None] + area[None, :] - inter
        over = inter / jnp.maximum(union, 1e-9) > THR
        ppr = pp[sl][:, None]
        idxr2 = lax.broadcasted_iota(jnp.int32, (rb, 1), 0) + blk * rb
        idxc2 = lax.broadcasted_iota(jnp.int32, (1, CAND), 1)
        cb = (ppr > pp[None, :]) | ((ppr == pp[None, :]) & (idxr2 < idxc2))
        c_scr[sl, :] = cb.astype(jnp.bfloat16)
        m_scr[sl, :] = (cb & over & (idxr2 < PRE)).astype(jnp.bfloat16)

    m = m_scr[...].astype(jnp.float32)
    cm = c_scr[...].astype(jnp.float32)
    validf = valid.astype(jnp.float32)

    def cond(st):
        kprev, k = st
        return jnp.any(k != kprev)

    def body(st):
        _, k = st
        sup = jnp.sum(m * k[:, None], axis=0)
        return k, jnp.where((sup == 0.0) & valid, 1.0, 0.0)

    _, k = lax.while_loop(cond, body, (validf - 1.0, validf))

    outpos = jnp.sum(cm * k[:, None], axis=0)
    nkrank = jnp.sum(cm * (1.0 - k)[:, None], axis=0)
    kept_n = jnp.sum(k)
    pos2 = jnp.where(k > 0, outpos, kept_n + nkrank).astype(jnp.int32)
    pos2_ref[0, 0, :] = pos2 + pl.program_id(0) * CAND
    keep = k > 0
    bv_ref[0, 0, :] = jnp.where(keep, y1, 0.0)
    bv_ref[0, 1, :] = jnp.where(keep, x1, 0.0)
    bv_ref[0, 2, :] = jnp.where(keep, y2, 0.0)
    bv_ref[0, 3, :] = jnp.where(keep, x2, 0.0)
    pv_ref[0, 0, :] = jnp.where(keep, pc, 0.0)


_k3 = pl.pallas_call(
    _k3_body,
    grid=(B,),
    in_specs=[
        pl.BlockSpec((1, 4, CAND), lambda i: (i, 0, 0)),
        pl.BlockSpec((1, 4, CAND), lambda i: (i, 0, 0)),
        pl.BlockSpec((1, 1, CAND), lambda i: (i, 0, 0)),
    ],
    out_specs=[
        pl.BlockSpec((1, 1, CAND), lambda i: (i, 0, 0)),
        pl.BlockSpec((1, 4, CAND), lambda i: (i, 0, 0)),
        pl.BlockSpec((1, 1, CAND), lambda i: (i, 0, 0)),
    ],
    out_shape=[
        jax.ShapeDtypeStruct((B, 1, CAND), jnp.int32),
        jax.ShapeDtypeStruct((B, 4, CAND), jnp.float32),
        jax.ShapeDtypeStruct((B, 1, CAND), jnp.float32),
    ],
    scratch_shapes=[
        pltpu.VMEM((CAND, CAND), jnp.bfloat16),
        pltpu.VMEM((CAND, CAND), jnp.bfloat16),
    ],
)


def _k4_body(pos3d_hbm, bv_flat_hbm, pv_hbm, o0, o1, o2, o3, os_,
             pbuf2, bbufs, sbuf, sem):
    w = lax.axis_index("s") * 2 + lax.axis_index("c")
    b = w // 8
    c = w % 8
    pltpu.sync_copy(pos3d_hbm.at[w], pbuf2)
    for col in range(4):
        src = b * 4 * CAND + col * CAND + c * 256
        pltpu.sync_copy(bv_flat_hbm.at[pl.ds(src, 256)], bbufs[col])
    pltpu.sync_copy(pv_hbm.at[pl.ds(w * 256, 256)], sbuf)
    ocols = [o0, o1, o2, o3]
    descs = []
    for j in range(2):
        for col in range(4):
            descs.append(pltpu.async_copy(
                bbufs[col].at[pl.ds(j * 128, 128)],
                ocols[col].at[pbuf2.at[j]], sem))
        descs.append(pltpu.async_copy(
            sbuf.at[pl.ds(j * 128, 128)], os_.at[pbuf2.at[j]], sem))
    for d in descs:
        d.wait()


@functools.cache
def _sc_kernels():
    mesh = _sc_mesh()
    k2a = pl.kernel(
        _k2a_body,
        out_type=jax.ShapeDtypeStruct((B * CSLOT,), jnp.int32),
        mesh=mesh,
        scratch_types=[
            pltpu.VMEM((K2A_ROWS, 128), jnp.int32),
            pltpu.VMEM((K2A_ROWS, 128), jnp.int32),
            pltpu.SemaphoreType.DMA,
        ],
    )
    k2b = pl.kernel(
        _k2b_body,
        out_type=[
            jax.ShapeDtypeStruct((B * 4 * CAND,), jnp.float32),
            jax.ShapeDtypeStruct((B * 4 * CAND,), jnp.float32),
            jax.ShapeDtypeStruct((B * CAND,), jnp.float32),
        ],
        mesh=mesh,
        scratch_types=[
            pltpu.VMEM((2, 128), jnp.int32),
            pltpu.VMEM((2, 128), jnp.int32),
            pltpu.VMEM((2, 128), jnp.int32),
            [pltpu.VMEM((256,), jnp.float32) for _ in range(4)],
            [pltpu.VMEM((256,), jnp.float32) for _ in range(4)],
            pltpu.VMEM((256,), jnp.float32),
            pltpu.SemaphoreType.DMA,
        ],
    )
    k4 = pl.kernel(
        _k4_body,
        out_type=[
            jax.ShapeDtypeStruct((B * CAND,), jnp.float32),
            jax.ShapeDtypeStruct((B * CAND,), jnp.float32),
            jax.ShapeDtypeStruct((B * CAND,), jnp.float32),
            jax.ShapeDtypeStruct((B * CAND,), jnp.float32),
            jax.ShapeDtypeStruct((B * CAND,), jnp.float32),
        ],
        mesh=mesh,
        scratch_types=[
            pltpu.VMEM((2, 128), jnp.int32),
            [pltpu.VMEM((256,), jnp.float32) for _ in range(4)],
            pltpu.VMEM((256,), jnp.float32),
            pltpu.SemaphoreType.DMA,
        ],
    )
    return k2a, k2b, k4


def kernel(rpn_bbox_deltas, rpn_labels, anchors):
    _k2a, _k2b, _k4 = _sc_kernels()
    labels_pad = jnp.pad(
        rpn_labels.reshape(B, N).astype(jnp.float32),
        ((0, 0), (0, NPAD - N)), constant_values=-jnp.inf)
    p, pos, aval = _k1(labels_pad)
    cand = _k2a(pos.reshape(NW, K2A_ROWS, 128), aval.reshape(NW, K2A_ROWS, 128))
    d = rpn_bbox_deltas.reshape(B, N, 4).astype(jnp.float32)
    a = anchors.astype(jnp.float32)
    cd, ca, cp = _k2b(
        cand.reshape(NW * 2, 2, 128),
        d[:, :, 0].reshape(-1), d[:, :, 1].reshape(-1),
        d[:, :, 2].reshape(-1), d[:, :, 3].reshape(-1),
        a[:, 0], a[:, 1], a[:, 2], a[:, 3],
        p.reshape(B * NPAD))
    pos2, bv, pv = _k3(
        cd.reshape(B, 4, CAND),
        ca.reshape(B, 4, CAND),
        cp.reshape(B, 1, CAND))
    o0, o1, o2, o3, os_ = _k4(
        pos2.reshape(NW, 2, 128),
        bv.reshape(B * 4 * CAND),
        pv.reshape(B * CAND))
    out_b = jnp.stack([o0, o1, o2, o3], axis=-1).reshape(B, CAND, 4)[:, :POST, :]
    out_s = os_.reshape(B, CAND)[:, :POST]
    return out_b, out_s


# speedup vs baseline: 192.4173x; 2.8177x over previous
import functools

import jax
import jax.numpy as jnp
from jax import lax
from jax.experimental import pallas as pl
from jax.experimental.pallas import tpu as pltpu
from jax.experimental.pallas import tpu_sc as plsc

B = 4
N = 20000
NPAD = 20480
CAND = 2048
CSLOT = 4096
PRE = 2000
POST = 1000
THR = 0.7
NW = 32
K2A_ROWS = (B * NPAD) // NW // 128

@functools.cache
def _sc_mesh():
    return plsc.VectorSubcoreMesh(core_axis_name="c", subcore_axis_name="s")


def _k1_body(x_ref, p_ref, pos_ref):
    x = x_ref[...]
    xm = jnp.max(x, axis=1, keepdims=True)
    e = jnp.exp(x - xm)
    s = jnp.sum(e, axis=1, keepdims=True)
    p = e / s
    p_ref[...] = p

    idx = lax.broadcasted_iota(jnp.int32, (B, NPAD), 1)

    def tstep(_, lohi):
        lo, hi = lohi
        mid = (lo + hi) // 2
        t = lax.bitcast_convert_type(mid, jnp.float32)
        cnt = jnp.sum((p > t).astype(jnp.int32), axis=1, keepdims=True)
        ge = cnt >= PRE
        return jnp.where(ge, mid, lo), jnp.where(ge, hi, mid)

    lo0 = jnp.zeros((B, 1), jnp.int32)
    hi0 = jnp.full((B, 1), 0x40000000, jnp.int32)
    _, hi = lax.fori_loop(0, 31, tstep, (lo0, hi0))
    t = lax.bitcast_convert_type(hi, jnp.float32)
    c_gt = jnp.sum((p > t).astype(jnp.int32), axis=1, keepdims=True)
    need = PRE - c_gt
    tie = p == t

    def mstep(_, lohi):
        lo, hi = lohi
        mid = (lo + hi) // 2
        cnt = jnp.sum((tie & (idx <= mid)).astype(jnp.int32), axis=1,
                      keepdims=True)
        ge = cnt >= need
        return jnp.where(ge, lo, mid), jnp.where(ge, mid, hi)

    mlo0 = jnp.full((B, 1), -1, jnp.int32)
    mhi0 = jnp.full((B, 1), NPAD - 1, jnp.int32)
    _, mhi = lax.fori_loop(0, 16, mstep, (mlo0, mhi0))
    sel = (p > t) | (tie & (idx <= mhi))

    r128 = lax.broadcasted_iota(jnp.int32, (128, 128), 0)
    c128 = lax.broadcasted_iota(jnp.int32, (128, 128), 1)
    tri_inc = (r128 <= c128).astype(jnp.float32)
    nblk = NPAD // 128
    rb = lax.broadcasted_iota(jnp.int32, (nblk, nblk), 0)
    cb = lax.broadcasted_iota(jnp.int32, (nblk, nblk), 1)
    tri_exc = (rb < cb).astype(jnp.float32)
    rows = []
    for b in range(B):
        sb = sel[b].astype(jnp.float32).reshape(nblk, 128)
        cinc = jnp.dot(sb, tri_inc, preferred_element_type=jnp.float32)
        bsum = cinc[:, 127]
        bpre = jnp.dot(bsum[None, :], tri_exc,
                       preferred_element_type=jnp.float32)[0]
        rows.append((bpre[:, None] + cinc - 1.0).reshape(NPAD))
    posl = jnp.stack(rows).astype(jnp.int32)
    dump = CAND + (idx % CAND)
    pos_ref[...] = jnp.where(sel, posl, dump)


_k1 = pl.pallas_call(
    _k1_body,
    out_shape=[
        jax.ShapeDtypeStruct((B, NPAD), jnp.float32),
        jax.ShapeDtypeStruct((B, NPAD), jnp.int32),
    ],
)


def _k2ab_body(pos_hbm, d0_hbm, d1_hbm, d2_hbm, d3_hbm,
               a0_hbm, a1_hbm, a2_hbm, a3_hbm, pflat_hbm,
               cd_flat, ca_flat, cp_flat,
               posbuf, valbuf, spc, candbuf,
               aidx, gidx, pidx, dbufs, abufs, pbuf, sem):
    c = lax.axis_index("c")
    s = lax.axis_index("s")
    w = c * 16 + s
    b = w // 8
    imgslot = s // 8
    cpart = s % 8
    pltpu.sync_copy(pos_hbm.at[w], posbuf)
    for j in range(K2A_ROWS):
        for t in range(8):
            pv = posbuf[j, pl.ds(t * 16, 16)] + imgslot * CSLOT
            posbuf[j, pl.ds(t * 16, 16)] = pv
            base = cpart * (NPAD // 8) + j * 128 + t * 16
            iv = base + lax.broadcasted_iota(jnp.int32, (16,), 0)
            valbuf[j, pl.ds(t * 16, 16)] = jnp.minimum(iv, N - 1)
    descs = [
        pltpu.async_copy(valbuf.at[j], spc.at[posbuf.at[j]], sem)
        for j in range(K2A_ROWS)
    ]
    for d in descs:
        d.wait()
    plsc.subcore_barrier()
    pltpu.sync_copy(spc.at[pl.ds(imgslot * CSLOT + cpart * 256, 256)], candbuf)
    for j in range(2):
        for t in range(8):
            v = candbuf[pl.ds(j * 128 + t * 16, 16)]
            v = jnp.minimum(jnp.maximum(v, 0), N - 1)
            aidx[j, pl.ds(t * 16, 16)] = v
            gidx[j, pl.ds(t * 16, 16)] = v + b * N
            pidx[j, pl.ds(t * 16, 16)] = v + b * NPAD
    dcols = [d0_hbm, d1_hbm, d2_hbm, d3_hbm]
    acols = [a0_hbm, a1_hbm, a2_hbm, a3_hbm]
    descs = []
    for j in range(2):
        for col in range(4):
            descs.append(pltpu.async_copy(
                dcols[col].at[gidx.at[j]],
                dbufs[col].at[pl.ds(j * 128, 128)], sem))
            descs.append(pltpu.async_copy(
                acols[col].at[aidx.at[j]],
                abufs[col].at[pl.ds(j * 128, 128)], sem))
        descs.append(pltpu.async_copy(
            pflat_hbm.at[pidx.at[j]], pbuf.at[pl.ds(j * 128, 128)], sem))
    for d in descs:
        d.wait()
    for col in range(4):
        o = b * 4 * CAND + col * CAND + cpart * 256
        pltpu.sync_copy(dbufs[col], cd_flat.at[pl.ds(o, 256)])
        pltpu.sync_copy(abufs[col], ca_flat.at[pl.ds(o, 256)])
    pltpu.sync_copy(pbuf, cp_flat.at[pl.ds(b * CAND + cpart * 256, 256)])


def _k3_body(d_ref, a_ref, p_ref, pos2_ref, bv_ref, pv_ref, m_scr, c_scr):
    d0 = d_ref[0, 0, :] * 0.1
    d1 = d_ref[0, 1, :] * 0.1
    d2 = d_ref[0, 2, :] * 0.2
    d3 = d_ref[0, 3, :] * 0.2
    a0 = a_ref[0, 0, :]
    a1 = a_ref[0, 1, :]
    a2 = a_ref[0, 2, :]
    a3 = a_ref[0, 3, :]
    w = a3 - a1
    h = a2 - a0
    cx = a1 + 0.5 * w
    cy = a0 + 0.5 * h
    bw = jnp.exp(d3) * w
    bh = jnp.exp(d2) * h
    bcx = d1 * w + cx
    bcy = d0 * h + cy
    y1 = bcy - 0.5 * bh
    x1 = bcx - 0.5 * bw
    y2 = bh + y1
    x2 = bw + x1
    area = (y2 - y1) * (x2 - x1)

    pc = p_ref[0, 0, :]
    idx = lax.broadcasted_iota(jnp.int32, (CAND,), 0)
    valid = idx < PRE
    pp = jnp.where(valid, pc, -1.0)

    rb = 256
    for blk in range(CAND // rb):
        sl = slice(blk * rb, (blk + 1) * rb)
        iy1 = jnp.maximum(y1[sl][:, None], y1[None, :])
        ix1 = jnp.maximum(x1[sl][:, None], x1[None, :])
        iy2 = jnp.minimum(y2[sl][:, None], y2[None, :])
        ix2 = jnp.minimum(x2[sl][:, None], x2[None, :])
        ih = jnp.clip(iy2 - iy1, 0.0)
        iw = jnp.clip(ix2 - ix1, 0.0)
        inter = ih * iw
        union = area[sl][:, None] + area[None, :] - inter
        over = inter / jnp.maximum(union, 1e-9) > THR
        ppr = pp[sl][:, None]
        idxr2 = lax.broadcasted_iota(jnp.int32, (rb, 1), 0) + blk * rb
        idxc2 = lax.broadcasted_iota(jnp.int32, (1, CAND), 1)
        cb = (ppr > pp[None, :]) | ((ppr == pp[None, :]) & (idxr2 < idxc2))
        c_scr[sl, :] = cb.astype(jnp.bfloat16)
        m_scr[sl, :] = (cb & over & (idxr2 < PRE)).astype(jnp.bfloat16)

    m = m_scr[...].astype(jnp.float32)
    cm = c_scr[...].astype(jnp.float32)
    validf = valid.astype(jnp.float32)

    def cond(st):
        kprev, k = st
        return jnp.any(k != kprev)

    def body(st):
        _, k = st
        sup = jnp.sum(m * k[:, None], axis=0)
        return k, jnp.where((sup == 0.0) & valid, 1.0, 0.0)

    _, k = lax.while_loop(cond, body, (validf - 1.0, validf))

    outpos = jnp.sum(cm * k[:, None], axis=0)
    nkrank = jnp.sum(cm * (1.0 - k)[:, None], axis=0)
    kept_n = jnp.sum(k)
    pos2 = jnp.where(k > 0, outpos, kept_n + nkrank).astype(jnp.int32)
    pos2_ref[0, 0, :] = pos2 + pl.program_id(0) * CAND
    keep = k > 0
    bv_ref[0, 0, :] = jnp.where(keep, y1, 0.0)
    bv_ref[0, 1, :] = jnp.where(keep, x1, 0.0)
    bv_ref[0, 2, :] = jnp.where(keep, y2, 0.0)
    bv_ref[0, 3, :] = jnp.where(keep, x2, 0.0)
    pv_ref[0, 0, :] = jnp.where(keep, pc, 0.0)


_k3 = pl.pallas_call(
    _k3_body,
    grid=(B,),
    in_specs=[
        pl.BlockSpec((1, 4, CAND), lambda i: (i, 0, 0)),
        pl.BlockSpec((1, 4, CAND), lambda i: (i, 0, 0)),
        pl.BlockSpec((1, 1, CAND), lambda i: (i, 0, 0)),
    ],
    out_specs=[
        pl.BlockSpec((1, 1, CAND), lambda i: (i, 0, 0)),
        pl.BlockSpec((1, 4, CAND), lambda i: (i, 0, 0)),
        pl.BlockSpec((1, 1, CAND), lambda i: (i, 0, 0)),
    ],
    out_shape=[
        jax.ShapeDtypeStruct((B, 1, CAND), jnp.int32),
        jax.ShapeDtypeStruct((B, 4, CAND), jnp.float32),
        jax.ShapeDtypeStruct((B, 1, CAND), jnp.float32),
    ],
    scratch_shapes=[
        pltpu.VMEM((CAND, CAND), jnp.bfloat16),
        pltpu.VMEM((CAND, CAND), jnp.bfloat16),
    ],
)


def _k4_body(pos3d_hbm, bv_flat_hbm, pv_hbm, o0, o1, o2, o3, os_,
             pbuf2, bbufs, sbuf, sem):
    w = lax.axis_index("s") * 2 + lax.axis_index("c")
    b = w // 8
    c = w % 8
    pltpu.sync_copy(pos3d_hbm.at[w], pbuf2)
    for col in range(4):
        src = b * 4 * CAND + col * CAND + c * 256
        pltpu.sync_copy(bv_flat_hbm.at[pl.ds(src, 256)], bbufs[col])
    pltpu.sync_copy(pv_hbm.at[pl.ds(w * 256, 256)], sbuf)
    ocols = [o0, o1, o2, o3]
    descs = []
    for j in range(2):
        for col in range(4):
            descs.append(pltpu.async_copy(
                bbufs[col].at[pl.ds(j * 128, 128)],
                ocols[col].at[pbuf2.at[j]], sem))
        descs.append(pltpu.async_copy(
            sbuf.at[pl.ds(j * 128, 128)], os_.at[pbuf2.at[j]], sem))
    for d in descs:
        d.wait()


@functools.cache
def _sc_kernels():
    mesh = _sc_mesh()
    k2ab = pl.kernel(
        _k2ab_body,
        out_type=[
            jax.ShapeDtypeStruct((B * 4 * CAND,), jnp.float32),
            jax.ShapeDtypeStruct((B * 4 * CAND,), jnp.float32),
            jax.ShapeDtypeStruct((B * CAND,), jnp.float32),
        ],
        mesh=mesh,
        scratch_types=[
            pltpu.VMEM((K2A_ROWS, 128), jnp.int32),
            pltpu.VMEM((K2A_ROWS, 128), jnp.int32),
            pltpu.VMEM_SHARED((2 * CSLOT,), jnp.int32),
            pltpu.VMEM((256,), jnp.int32),
            pltpu.VMEM((2, 128), jnp.int32),
            pltpu.VMEM((2, 128), jnp.int32),
            pltpu.VMEM((2, 128), jnp.int32),
            [pltpu.VMEM((256,), jnp.float32) for _ in range(4)],
            [pltpu.VMEM((256,), jnp.float32) for _ in range(4)],
            pltpu.VMEM((256,), jnp.float32),
            pltpu.SemaphoreType.DMA,
        ],
    )
    k4 = pl.kernel(
        _k4_body,
        out_type=[
            jax.ShapeDtypeStruct((B * CAND,), jnp.float32),
            jax.ShapeDtypeStruct((B * CAND,), jnp.float32),
            jax.ShapeDtypeStruct((B * CAND,), jnp.float32),
            jax.ShapeDtypeStruct((B * CAND,), jnp.float32),
            jax.ShapeDtypeStruct((B * CAND,), jnp.float32),
        ],
        mesh=mesh,
        scratch_types=[
            pltpu.VMEM((2, 128), jnp.int32),
            [pltpu.VMEM((256,), jnp.float32) for _ in range(4)],
            pltpu.VMEM((256,), jnp.float32),
            pltpu.SemaphoreType.DMA,
        ],
    )
    return k2ab, k4


def kernel(rpn_bbox_deltas, rpn_labels, anchors):
    _k2ab, _k4 = _sc_kernels()
    labels_pad = jnp.pad(
        rpn_labels.reshape(B, N).astype(jnp.float32),
        ((0, 0), (0, NPAD - N)), constant_values=-jnp.inf)
    p, pos = _k1(labels_pad)
    d = rpn_bbox_deltas.reshape(B, N, 4).astype(jnp.float32)
    a = anchors.astype(jnp.float32)
    cd, ca, cp = _k2ab(
        pos.reshape(NW, K2A_ROWS, 128),
        d[:, :, 0].reshape(-1), d[:, :, 1].reshape(-1),
        d[:, :, 2].reshape(-1), d[:, :, 3].reshape(-1),
        a[:, 0], a[:, 1], a[:, 2], a[:, 3],
        p.reshape(B * NPAD))
    pos2, bv, pv = _k3(
        cd.reshape(B, 4, CAND),
        ca.reshape(B, 4, CAND),
        cp.reshape(B, 1, CAND))
    o0, o1, o2, o3, os_ = _k4(
        pos2.reshape(NW, 2, 128),
        bv.reshape(B * 4 * CAND),
        pv.reshape(B * CAND))
    out_b = jnp.stack([o0, o1, o2, o3], axis=-1).reshape(B, CAND, 4)[:, :POST, :]
    out_s = os_.reshape(B, CAND)[:, :POST]
    return out_b, out_s


# speedup vs baseline: 303.0781x; 1.5751x over previous
import functools

import jax
import jax.numpy as jnp
from jax import lax
from jax.experimental import pallas as pl
from jax.experimental.pallas import tpu as pltpu
from jax.experimental.pallas import tpu_sc as plsc

B = 4
N = 20000
NPAD = 20480
CAND = 2048
CSLOT = 4096
PRE = 2000
POST = 1000
THR = 0.7
NW = 32
K2A_ROWS = (B * NPAD) // NW // 128

@functools.cache
def _sc_mesh():
    return plsc.VectorSubcoreMesh(core_axis_name="c", subcore_axis_name="s")


def _k1_body(x_ref, p_ref, pos_ref):
    x = x_ref[...]
    xm = jnp.max(x, axis=1, keepdims=True)
    e = jnp.exp(x - xm)
    s = jnp.sum(e, axis=1, keepdims=True)
    p = e / s
    p_ref[...] = p

    idx = lax.broadcasted_iota(jnp.int32, (B, NPAD), 1)

    def tstep(_, lohi):
        lo, hi = lohi
        mid = (lo + hi) // 2
        t = lax.bitcast_convert_type(mid, jnp.float32)
        cnt = jnp.sum((p > t).astype(jnp.int32), axis=1, keepdims=True)
        ge = cnt >= PRE
        return jnp.where(ge, mid, lo), jnp.where(ge, hi, mid)

    lo0 = jnp.zeros((B, 1), jnp.int32)
    hi0 = jnp.full((B, 1), 0x40000000, jnp.int32)
    _, hi = lax.fori_loop(0, 31, tstep, (lo0, hi0))
    t = lax.bitcast_convert_type(hi, jnp.float32)
    c_gt = jnp.sum((p > t).astype(jnp.int32), axis=1, keepdims=True)
    need = PRE - c_gt
    tie = p == t

    def mstep(_, lohi):
        lo, hi = lohi
        mid = (lo + hi) // 2
        cnt = jnp.sum((tie & (idx <= mid)).astype(jnp.int32), axis=1,
                      keepdims=True)
        ge = cnt >= need
        return jnp.where(ge, lo, mid), jnp.where(ge, mid, hi)

    mlo0 = jnp.full((B, 1), -1, jnp.int32)
    mhi0 = jnp.full((B, 1), NPAD - 1, jnp.int32)
    _, mhi = lax.fori_loop(0, 16, mstep, (mlo0, mhi0))
    sel = (p > t) | (tie & (idx <= mhi))

    r128 = lax.broadcasted_iota(jnp.int32, (128, 128), 0)
    c128 = lax.broadcasted_iota(jnp.int32, (128, 128), 1)
    tri_inc = (r128 <= c128).astype(jnp.float32)
    nblk = NPAD // 128
    rb = lax.broadcasted_iota(jnp.int32, (nblk, nblk), 0)
    cb = lax.broadcasted_iota(jnp.int32, (nblk, nblk), 1)
    tri_exc = (rb < cb).astype(jnp.float32)
    rows = []
    for b in range(B):
        sb = sel[b].astype(jnp.float32).reshape(nblk, 128)
        cinc = jnp.dot(sb, tri_inc, preferred_element_type=jnp.float32)
        bsum = cinc[:, 127]
        bpre = jnp.dot(bsum[None, :], tri_exc,
                       preferred_element_type=jnp.float32)[0]
        rows.append((bpre[:, None] + cinc - 1.0).reshape(NPAD))
    posl = jnp.stack(rows).astype(jnp.int32)
    dump = CAND + (idx % CAND)
    pos_ref[...] = jnp.where(sel, posl, dump)


_k1 = pl.pallas_call(
    _k1_body,
    out_shape=[
        jax.ShapeDtypeStruct((B, NPAD), jnp.float32),
        jax.ShapeDtypeStruct((B, NPAD), jnp.int32),
    ],
)


def _k2ab_body(pos_hbm, d0_hbm, d1_hbm, d2_hbm, d3_hbm,
               a0_hbm, a1_hbm, a2_hbm, a3_hbm, pflat_hbm,
               cd_flat, ca_flat, cp_flat,
               posbuf, valbuf, spc, candbuf,
               aidx, gidx, pidx, dbufs, abufs, pbuf, sem):
    c = lax.axis_index("c")
    s = lax.axis_index("s")
    w = c * 16 + s
    b = w // 8
    imgslot = s // 8
    cpart = s % 8
    pltpu.sync_copy(pos_hbm.at[w], posbuf)
    for j in range(K2A_ROWS):
        for t in range(8):
            pv = posbuf[j, pl.ds(t * 16, 16)] + imgslot * CSLOT
            posbuf[j, pl.ds(t * 16, 16)] = pv
            base = cpart * (NPAD // 8) + j * 128 + t * 16
            iv = base + lax.broadcasted_iota(jnp.int32, (16,), 0)
            valbuf[j, pl.ds(t * 16, 16)] = jnp.minimum(iv, N - 1)
    descs = [
        pltpu.async_copy(valbuf.at[j], spc.at[posbuf.at[j]], sem)
        for j in range(K2A_ROWS)
    ]
    for d in descs:
        d.wait()
    plsc.subcore_barrier()
    pltpu.sync_copy(spc.at[pl.ds(imgslot * CSLOT + cpart * 256, 256)], candbuf)
    for j in range(2):
        for t in range(8):
            v = candbuf[pl.ds(j * 128 + t * 16, 16)]
            v = jnp.minimum(jnp.maximum(v, 0), N - 1)
            aidx[j, pl.ds(t * 16, 16)] = v
            gidx[j, pl.ds(t * 16, 16)] = v + b * N
            pidx[j, pl.ds(t * 16, 16)] = v + b * NPAD
    dcols = [d0_hbm, d1_hbm, d2_hbm, d3_hbm]
    acols = [a0_hbm, a1_hbm, a2_hbm, a3_hbm]
    descs = []
    for j in range(2):
        for col in range(4):
            descs.append(pltpu.async_copy(
                dcols[col].at[gidx.at[j]],
                dbufs[col].at[pl.ds(j * 128, 128)], sem))
            descs.append(pltpu.async_copy(
                acols[col].at[aidx.at[j]],
                abufs[col].at[pl.ds(j * 128, 128)], sem))
        descs.append(pltpu.async_copy(
            pflat_hbm.at[pidx.at[j]], pbuf.at[pl.ds(j * 128, 128)], sem))
    for d in descs:
        d.wait()
    for col in range(4):
        o = b * 4 * CAND + col * CAND + cpart * 256
        pltpu.sync_copy(dbufs[col], cd_flat.at[pl.ds(o, 256)])
        pltpu.sync_copy(abufs[col], ca_flat.at[pl.ds(o, 256)])
    pltpu.sync_copy(pbuf, cp_flat.at[pl.ds(b * CAND + cpart * 256, 256)])


def _k3_body(d_ref, a_ref, p_ref, pos2_ref, bv_ref, pv_ref, m_scr, c_scr):
    d0 = d_ref[0, 0, :] * 0.1
    d1 = d_ref[0, 1, :] * 0.1
    d2 = d_ref[0, 2, :] * 0.2
    d3 = d_ref[0, 3, :] * 0.2
    a0 = a_ref[0, 0, :]
    a1 = a_ref[0, 1, :]
    a2 = a_ref[0, 2, :]
    a3 = a_ref[0, 3, :]
    w = a3 - a1
    h = a2 - a0
    cx = a1 + 0.5 * w
    cy = a0 + 0.5 * h
    bw = jnp.exp(d3) * w
    bh = jnp.exp(d2) * h
    bcx = d1 * w + cx
    bcy = d0 * h + cy
    y1 = bcy - 0.5 * bh
    x1 = bcx - 0.5 * bw
    y2 = bh + y1
    x2 = bw + x1
    area = (y2 - y1) * (x2 - x1)

    pc = p_ref[0, 0, :]
    idx = lax.broadcasted_iota(jnp.int32, (CAND,), 0)
    valid = idx < PRE
    pp = jnp.where(valid, pc, -1.0)

    rb = 256
    for blk in range(CAND // rb):
        sl = slice(blk * rb, (blk + 1) * rb)
        iy1 = jnp.maximum(y1[sl][:, None], y1[None, :])
        ix1 = jnp.maximum(x1[sl][:, None], x1[None, :])
        iy2 = jnp.minimum(y2[sl][:, None], y2[None, :])
        ix2 = jnp.minimum(x2[sl][:, None], x2[None, :])
        ih = jnp.clip(iy2 - iy1, 0.0)
        iw = jnp.clip(ix2 - ix1, 0.0)
        inter = ih * iw
        union = area[sl][:, None] + area[None, :] - inter
        over = inter / jnp.maximum(union, 1e-9) > THR
        ppr = pp[sl][:, None]
        idxr2 = lax.broadcasted_iota(jnp.int32, (rb, 1), 0) + blk * rb
        idxc2 = lax.broadcasted_iota(jnp.int32, (1, CAND), 1)
        cb = (ppr > pp[None, :]) | ((ppr == pp[None, :]) & (idxr2 < idxc2))
        c_scr[sl, :] = cb.astype(jnp.bfloat16)
        m_scr[sl, :] = (cb & over & (idxr2 < PRE)).astype(jnp.bfloat16)

    m = m_scr[...].astype(jnp.float32)
    cm = c_scr[...].astype(jnp.float32)
    validf = valid.astype(jnp.float32)

    def cond(st):
        kprev, k = st
        return jnp.any(k != kprev)

    def body(st):
        _, k = st
        sup = jnp.sum(m * k[:, None], axis=0)
        return k, jnp.where((sup == 0.0) & valid, 1.0, 0.0)

    _, k = lax.while_loop(cond, body, (validf - 1.0, validf))

    outpos = jnp.sum(cm * k[:, None], axis=0)
    nkrank = jnp.sum(cm * (1.0 - k)[:, None], axis=0)
    kept_n = jnp.sum(k)
    pos2 = jnp.where(k > 0, outpos, kept_n + nkrank).astype(jnp.int32)
    pos2_ref[0, 0, :] = pos2
    keep = k > 0
    bv_ref[0, 0, :] = jnp.where(keep, y1, 0.0)
    bv_ref[0, 1, :] = jnp.where(keep, x1, 0.0)
    bv_ref[0, 2, :] = jnp.where(keep, y2, 0.0)
    bv_ref[0, 3, :] = jnp.where(keep, x2, 0.0)
    pv_ref[0, 0, :] = jnp.where(keep, pc, 0.0)


_k3 = pl.pallas_call(
    _k3_body,
    grid=(B,),
    in_specs=[
        pl.BlockSpec((1, 4, CAND), lambda i: (i, 0, 0)),
        pl.BlockSpec((1, 4, CAND), lambda i: (i, 0, 0)),
        pl.BlockSpec((1, 1, CAND), lambda i: (i, 0, 0)),
    ],
    out_specs=[
        pl.BlockSpec((1, 1, CAND), lambda i: (i, 0, 0)),
        pl.BlockSpec((1, 4, CAND), lambda i: (i, 0, 0)),
        pl.BlockSpec((1, 1, CAND), lambda i: (i, 0, 0)),
    ],
    out_shape=[
        jax.ShapeDtypeStruct((B, 1, CAND), jnp.int32),
        jax.ShapeDtypeStruct((B, 4, CAND), jnp.float32),
        jax.ShapeDtypeStruct((B, 1, CAND), jnp.float32),
    ],
    scratch_shapes=[
        pltpu.VMEM((CAND, CAND), jnp.bfloat16),
        pltpu.VMEM((CAND, CAND), jnp.bfloat16),
    ],
)


def _k4_body(pos3d_hbm, bv_flat_hbm, pv_hbm, o0, o1, o2, o3, os_,
             pbuf2, bbufs, sbuf, pcols, spk, sem):
    c = lax.axis_index("c")
    s = lax.axis_index("s")
    w = c * 16 + s
    b = w // 8
    imgslot = s // 8
    cpart = s % 8
    pltpu.sync_copy(pos3d_hbm.at[w], pbuf2)
    for col in range(4):
        src = b * 4 * CAND + col * CAND + cpart * 256
        pltpu.sync_copy(bv_flat_hbm.at[pl.ds(src, 256)], bbufs[col])
    pltpu.sync_copy(pv_hbm.at[pl.ds(w * 256, 256)], sbuf)
    for j in range(2):
        for t in range(8):
            pv = pbuf2[j, pl.ds(t * 16, 16)]
            for col in range(5):
                pcols[col][j, pl.ds(t * 16, 16)] = (
                    pv + (imgslot * 5 + col) * CAND)
    descs = []
    for j in range(2):
        for col in range(4):
            descs.append(pltpu.async_copy(
                bbufs[col].at[pl.ds(j * 128, 128)],
                spk.at[pcols[col].at[j]], sem))
        descs.append(pltpu.async_copy(
            sbuf.at[pl.ds(j * 128, 128)], spk.at[pcols[4].at[j]], sem))
    for d in descs:
        d.wait()
    plsc.subcore_barrier()
    ocols = [o0, o1, o2, o3, os_]
    for r in range(10):
        @pl.when(s == r)
        def _copyout(r=r):
            pltpu.sync_copy(
                spk.at[pl.ds(r * CAND, CAND)],
                ocols[r % 5].at[pl.ds((c * 2 + r // 5) * CAND, CAND)])


@functools.cache
def _sc_kernels():
    mesh = _sc_mesh()
    k2ab = pl.kernel(
        _k2ab_body,
        out_type=[
            jax.ShapeDtypeStruct((B * 4 * CAND,), jnp.float32),
            jax.ShapeDtypeStruct((B * 4 * CAND,), jnp.float32),
            jax.ShapeDtypeStruct((B * CAND,), jnp.float32),
        ],
        mesh=mesh,
        scratch_types=[
            pltpu.VMEM((K2A_ROWS, 128), jnp.int32),
            pltpu.VMEM((K2A_ROWS, 128), jnp.int32),
            pltpu.VMEM_SHARED((2 * CSLOT,), jnp.int32),
            pltpu.VMEM((256,), jnp.int32),
            pltpu.VMEM((2, 128), jnp.int32),
            pltpu.VMEM((2, 128), jnp.int32),
            pltpu.VMEM((2, 128), jnp.int32),
            [pltpu.VMEM((256,), jnp.float32) for _ in range(4)],
            [pltpu.VMEM((256,), jnp.float32) for _ in range(4)],
            pltpu.VMEM((256,), jnp.float32),
            pltpu.SemaphoreType.DMA,
        ],
    )
    k4 = pl.kernel(
        _k4_body,
        out_type=[
            jax.ShapeDtypeStruct((B * CAND,), jnp.float32),
            jax.ShapeDtypeStruct((B * CAND,), jnp.float32),
            jax.ShapeDtypeStruct((B * CAND,), jnp.float32),
            jax.ShapeDtypeStruct((B * CAND,), jnp.float32),
            jax.ShapeDtypeStruct((B * CAND,), jnp.float32),
        ],
        mesh=mesh,
        scratch_types=[
            pltpu.VMEM((2, 128), jnp.int32),
            [pltpu.VMEM((256,), jnp.float32) for _ in range(4)],
            pltpu.VMEM((256,), jnp.float32),
            [pltpu.VMEM((2, 128), jnp.int32) for _ in range(5)],
            pltpu.VMEM_SHARED((2 * 5 * CAND,), jnp.float32),
            pltpu.SemaphoreType.DMA,
        ],
    )
    return k2ab, k4


def kernel(rpn_bbox_deltas, rpn_labels, anchors):
    _k2ab, _k4 = _sc_kernels()
    labels_pad = jnp.pad(
        rpn_labels.reshape(B, N).astype(jnp.float32),
        ((0, 0), (0, NPAD - N)), constant_values=-jnp.inf)
    p, pos = _k1(labels_pad)
    d = rpn_bbox_deltas.reshape(B, N, 4).astype(jnp.float32)
    a = anchors.astype(jnp.float32)
    cd, ca, cp = _k2ab(
        pos.reshape(NW, K2A_ROWS, 128),
        d[:, :, 0].reshape(-1), d[:, :, 1].reshape(-1),
        d[:, :, 2].reshape(-1), d[:, :, 3].reshape(-1),
        a[:, 0], a[:, 1], a[:, 2], a[:, 3],
        p.reshape(B * NPAD))
    pos2, bv, pv = _k3(
        cd.reshape(B, 4, CAND),
        ca.reshape(B, 4, CAND),
        cp.reshape(B, 1, CAND))
    o0, o1, o2, o3, os_ = _k4(
        pos2.reshape(NW, 2, 128),
        bv.reshape(B * 4 * CAND),
        pv.reshape(B * CAND))
    out_b = jnp.stack([o0, o1, o2, o3], axis=-1).reshape(B, CAND, 4)[:, :POST, :]
    out_s = os_.reshape(B, CAND)[:, :POST]
    return out_b, out_s
